# baseline (speedup 1.0000x reference)
import functools

import jax
import jax.numpy as jnp
from jax import lax
from jax.experimental import pallas as pl
from jax.experimental.pallas import tpu as pltpu
from jax.experimental.pallas import tpu_sc as plsc

_N_CLS = 4096
_N_CTX = 4
_D = 512
_CTX_LEN = 77
_N_GATHER = _CTX_LEN - _N_CTX
_IDX_PAD = 80
_NC = 2
_NS = 16
_NW = _NC * _NS
_CPW = _N_CLS // _NW


def _make_prompt_kernel():
    mesh = plsc.VectorSubcoreMesh(core_axis_name="c", subcore_axis_name="s")

    @functools.partial(
        pl.kernel,
        mesh=mesh,
        out_type=jax.ShapeDtypeStruct((_N_CLS, _CTX_LEN, _D), jnp.float32),
        scratch_types=[
            pltpu.VMEM((_CPW, _IDX_PAD), jnp.int32),
            pltpu.VMEM((_N_CTX, _D), jnp.float32),
            pltpu.VMEM((_IDX_PAD, _D), jnp.float32),
            pltpu.SemaphoreType.DMA,
        ],
        compiler_params=pltpu.CompilerParams(use_tc_tiling_on_sc=False),
    )
    def prompt_kernel(idx_hbm, ctx_hbm, table_hbm, out_hbm, idx_v, ctx_v, buf_v, sem):
        wid = lax.axis_index("s") * _NC + lax.axis_index("c")
        base = wid * _CPW
        pltpu.sync_copy(idx_hbm.at[pl.ds(base, _CPW)], idx_v)
        pltpu.sync_copy(ctx_hbm, ctx_v)

        def body(i, carry):
            c = base + i
            pltpu.async_copy(table_hbm.at[idx_v.at[i]], buf_v, sem).wait()
            pltpu.sync_copy(buf_v.at[pl.ds(0, 1)], out_hbm.at[c, pl.ds(0, 1)])
            pltpu.sync_copy(ctx_v, out_hbm.at[c, pl.ds(1, _N_CTX)])
            pltpu.sync_copy(
                buf_v.at[pl.ds(1, _N_GATHER - 1)],
                out_hbm.at[c, pl.ds(1 + _N_CTX, _N_GATHER - 1)],
            )
            return carry

        lax.fori_loop(0, _CPW, body, 0)

    return prompt_kernel


_prompt_kernel = _make_prompt_kernel()


def kernel(tokenized_prompts, ctx, token_embedding):
    idx = jnp.pad(
        tokenized_prompts[:, :_N_GATHER], ((0, 0), (0, _IDX_PAD - _N_GATHER))
    )
    prompts = _prompt_kernel(idx, ctx, token_embedding)
    return (prompts, tokenized_prompts)
